# baseline (speedup 1.0000x reference)
import functools
import math

import numpy as np
import jax
import jax.numpy as jnp
from jax import lax
from jax.experimental import pallas as pl
from jax.experimental.pallas import tpu as pltpu
from jax.experimental.pallas import tpu_sc as plsc

DIM = 128
BITS = 4
EPS = 1e-08
L = 16
VPR = DIM // L
NC, NS = 2, 16
NW = NC * NS


def _lm_centroids(bits, iters=100):
    n = 2 ** bits
    xs = np.linspace(-8.0, 8.0, 200001)
    pdf = np.exp(-0.5 * xs ** 2)
    cdf = np.cumsum(pdf)
    cdf = cdf / cdf[-1]
    c = np.interp((np.arange(n) + 0.5) / n, cdf, xs)
    for _ in range(iters):
        b = 0.5 * (c[:-1] + c[1:])
        idx = np.searchsorted(b, xs)
        num = np.bincount(idx, weights=pdf * xs, minlength=n)
        den = np.bincount(idx, weights=pdf, minlength=n)
        c = np.where(den > 1e-12, num / np.maximum(den, 1e-12), c)
    return c.astype(np.float32)


_CEN = _lm_centroids(BITS)
_BND = (0.5 * (_CEN[:-1] + _CEN[1:])).astype(np.float32)
_BND16 = np.concatenate([_BND, [np.float32(np.inf)]])
_MAXC = float(_CEN.max())
_SGN = (np.random.RandomState(42).randint(0, 2, (1, DIM)) * 2 - 1).astype(
    np.float32)[0]
_S = 1.0 / math.sqrt(float(DIM))

_XMIN = np.float32(-3.046875)
_XMAX = np.float32(2.859375)
_INVD = np.float32(32.0 / 3.0)
_XOFF = np.float32(-_XMIN * _INVD)


def _build_lut():
    edges = (float(_XMIN) + (3.0 / 32.0) * np.arange(65)).astype(np.float64)
    lo = np.zeros(64, np.float32)
    hi = np.zeros(64, np.float32)
    bd = np.zeros(64, np.float32)
    bnd = _BND.astype(np.float64)
    for k in range(64):
        base = int(np.searchsorted(bnd, edges[k], side="left"))
        inside = np.where((bnd >= edges[k]) & (bnd < edges[k + 1]))[0]
        assert len(inside) <= 1
        assert np.abs(bnd[:, None] - edges[None, :]).min() > 1e-4
        if len(inside):
            assert inside[0] == base
            lo[k] = _CEN[base]
            hi[k] = _CEN[base + 1]
            bd[k] = _BND[base]
        else:
            lo[k] = hi[k] = _CEN[min(base, 15)]
            bd[k] = np.inf
    return lo, hi, bd


_LUT_LO, _LUT_HI, _LUT_BD = _build_lut()

_CONSTS = np.concatenate(
    [_CEN, _BND16, (_SGN * np.float32(_S)).astype(np.float32),
     _LUT_LO, _LUT_HI, _LUT_BD]
).astype(np.float32)

_GDN = lax.GatherDimensionNumbers(
    offset_dims=(), collapsed_slice_dims=(0,), start_index_map=(0,))


def _dg(vec, idx):
    return lax.gather(vec, idx[:, None], _GDN, (1,),
                      mode=lax.GatherScatterMode.PROMISE_IN_BOUNDS)


def _lane_sum(v):
    lane = lax.iota(jnp.int32, L)
    for h in (1, 2, 4, 8):
        v = v + _dg(v, lane ^ h)
    return v


def _lane_max(v):
    lane = lax.iota(jnp.int32, L)
    for h in (1, 2, 4, 8):
        v = jnp.maximum(v, _dg(v, lane ^ h))
    return v


def _fwht_regs(v):
    lane = lax.iota(jnp.int32, L)
    for h in (1, 2, 4, 8):
        perm = lane ^ h
        pm = jnp.where((lane & h) == 0, 1.0, -1.0).astype(jnp.float32)
        v = [_dg(x, perm) + pm * x for x in v]
    for hb in (1, 2, 4):
        nv = list(v)
        for j in range(VPR):
            if j & hb == 0:
                nv[j] = v[j] + v[j ^ hb]
            else:
                nv[j] = v[j ^ hb] - v[j]
        v = nv
    return v


def _quant_pass(u, q, bvec, cvec):
    recon = []
    num_acc = None
    den_acc = None
    for j in range(VPR):
        xq = u[j] * q
        idx = jnp.zeros((L,), jnp.int32)
        for stp in (8, 4, 2, 1):
            bv = _dg(bvec, idx + (stp - 1))
            idx = idx + jnp.where(bv < xq, stp, 0).astype(jnp.int32)
        r = _dg(cvec, idx)
        recon.append(r)
        nj = u[j] * r
        dj = r * r
        num_acc = nj if num_acc is None else num_acc + nj
        den_acc = dj if den_acc is None else den_acc + dj
    return recon, _lane_sum(num_acc), _lane_sum(den_acc)


def _sqrt_newton(ss):
    bits = lax.bitcast_convert_type(ss, jnp.int32)
    y = lax.bitcast_convert_type(0x5F3759DF - (bits >> 1), jnp.float32)
    for _ in range(3):
        y = y * (1.5 - 0.5 * ss * y * y)
    return ss * y


def _make_sc_call(rows, ch, interpret=False):
    rpw = rows // NW
    nch = rpw // ch
    mesh = plsc.VectorSubcoreMesh(core_axis_name="c", subcore_axis_name="s",
                                  num_cores=NC, num_subcores=NS)

    @functools.partial(
        pl.kernel,
        out_type=jax.ShapeDtypeStruct((rows * DIM,), jnp.float32),
        mesh=mesh,
        scratch_types=[
            pltpu.VMEM((_CONSTS.size,), jnp.float32),
            pltpu.VMEM((ch * DIM,), jnp.float32),
            pltpu.VMEM((ch * DIM,), jnp.float32),
        ],
        interpret=interpret,
    )
    def sc_fn(x_hbm, c_hbm, o_hbm, cbuf, inb, outb):
        wid = lax.axis_index("s") * NC + lax.axis_index("c")
        base = wid * rpw * DIM

        pltpu.sync_copy(c_hbm, cbuf)
        cvec = cbuf[pl.ds(0, L)]
        bvec = cbuf[pl.ds(L, L)]
        sgn_s = [cbuf[pl.ds(2 * L + L * j, L)] for j in range(VPR)]

        def row_body(i):
            ro = i * DIM
            v = [inb[pl.ds(ro + L * j, L)] for j in range(VPR)]
            ssa = v[0] * v[0]
            for j in range(1, VPR):
                ssa = ssa + v[j] * v[j]
            ss = _lane_sum(ssa)
            u = _fwht_regs([v[j] * sgn_s[j] for j in range(VPR)])
            au = [jnp.abs(t) for t in u]
            mx = au[0]
            sa = au[0]
            for j in range(1, VPR):
                mx = jnp.maximum(mx, au[j])
                sa = sa + au[j]
            maxu = _lane_max(mx)
            sumau = _lane_sum(sa)

            norm = _sqrt_newton(ss) + EPS
            t = EPS * norm
            q1 = 1.0 / (maxu * (1.0 / _MAXC) + t)
            rec_u, num1, den1 = _quant_pass(u, q1, bvec, cvec)
            g1p = num1 / (den1 + EPS)
            q2 = 1.0 / (g1p + t)
            rec_f, num2, den2 = _quant_pass(u, q2, bvec, cvec)

            g2 = num2 / (den2 + EPS)
            grms = maxu * (1.0 / _MAXC)
            spiky = maxu > 5.0 * (sumau * (1.0 / DIM) + t)
            gain = jnp.where(spiky, grms, g2)
            rec = [jnp.where(spiky, rec_u[j], rec_f[j]) for j in range(VPR)]

            w = _fwht_regs(rec)
            for j in range(VPR):
                outb[pl.ds(ro + L * j, L)] = w[j] * sgn_s[j] * gain

        def chunk_body(ci, carry):
            off = base + ci * (ch * DIM)
            pltpu.sync_copy(x_hbm.at[pl.ds(off, ch * DIM)], inb)
            plsc.parallel_loop(0, ch, 1, unroll=4)(row_body)
            pltpu.sync_copy(outb, o_hbm.at[pl.ds(off, ch * DIM)])
            return carry

        lax.fori_loop(0, nch, chunk_body, 0)

    return sc_fn


def kernel(x):
    shape = x.shape
    rows = x.size // DIM
    xf = x.astype(jnp.float32).reshape(rows * DIM)
    rpw = rows // NW
    ch = 64 if rpw % 64 == 0 else rpw
    out = _make_sc_call(rows, ch)(xf, jnp.asarray(_CONSTS))
    return out.reshape(shape)


# speedup vs baseline: 1.6598x; 1.6598x over previous
import functools
import math

import numpy as np
import jax
import jax.numpy as jnp
from jax import lax
from jax.experimental import pallas as pl
from jax.experimental.pallas import tpu as pltpu
from jax.experimental.pallas import tpu_sc as plsc

DIM = 128
BITS = 4
EPS = 1e-08
L = 16
VPR = DIM // L
NC, NS = 2, 16
NW = NC * NS


def _lm_centroids(bits, iters=100):
    n = 2 ** bits
    xs = np.linspace(-8.0, 8.0, 200001)
    pdf = np.exp(-0.5 * xs ** 2)
    cdf = np.cumsum(pdf)
    cdf = cdf / cdf[-1]
    c = np.interp((np.arange(n) + 0.5) / n, cdf, xs)
    for _ in range(iters):
        b = 0.5 * (c[:-1] + c[1:])
        idx = np.searchsorted(b, xs)
        num = np.bincount(idx, weights=pdf * xs, minlength=n)
        den = np.bincount(idx, weights=pdf, minlength=n)
        c = np.where(den > 1e-12, num / np.maximum(den, 1e-12), c)
    return c.astype(np.float32)


_CEN = _lm_centroids(BITS)
_BND = (0.5 * (_CEN[:-1] + _CEN[1:])).astype(np.float32)
_BND16 = np.concatenate([_BND, [np.float32(np.inf)]])
_MAXC = float(_CEN.max())
_SGN = (np.random.RandomState(42).randint(0, 2, (1, DIM)) * 2 - 1).astype(
    np.float32)[0]
_S = 1.0 / math.sqrt(float(DIM))

_XMIN = np.float32(-3.046875)
_XMAX = np.float32(2.859375)
_INVD = np.float32(32.0 / 3.0)
_XOFF = np.float32(-_XMIN * _INVD)


def _build_lut():
    edges = (float(_XMIN) + (3.0 / 32.0) * np.arange(65)).astype(np.float64)
    lo = np.zeros(64, np.float32)
    hi = np.zeros(64, np.float32)
    bd = np.zeros(64, np.float32)
    bnd = _BND.astype(np.float64)
    for k in range(64):
        base = int(np.searchsorted(bnd, edges[k], side="left"))
        inside = np.where((bnd >= edges[k]) & (bnd < edges[k + 1]))[0]
        assert len(inside) <= 1
        assert np.abs(bnd[:, None] - edges[None, :]).min() > 1e-4
        if len(inside):
            assert inside[0] == base
            lo[k] = _CEN[base]
            hi[k] = _CEN[base + 1]
            bd[k] = _BND[base]
        else:
            lo[k] = hi[k] = _CEN[min(base, 15)]
            bd[k] = np.inf
    return lo, hi, bd


_LUT_LO, _LUT_HI, _LUT_BD = _build_lut()

_CONSTS = np.concatenate(
    [_CEN, _BND16, (_SGN * np.float32(_S)).astype(np.float32),
     _LUT_LO, _LUT_HI, _LUT_BD]
).astype(np.float32)

_GDN = lax.GatherDimensionNumbers(
    offset_dims=(), collapsed_slice_dims=(0,), start_index_map=(0,))


def _dg(vec, idx):
    return lax.gather(vec, idx[:, None], _GDN, (1,),
                      mode=lax.GatherScatterMode.PROMISE_IN_BOUNDS)


def _lane_sum(v):
    lane = lax.iota(jnp.int32, L)
    for h in (1, 2, 4, 8):
        v = v + _dg(v, lane ^ h)
    return v


def _lane_max(v):
    lane = lax.iota(jnp.int32, L)
    for h in (1, 2, 4, 8):
        v = jnp.maximum(v, _dg(v, lane ^ h))
    return v


def _fwht_regs(v):
    lane = lax.iota(jnp.int32, L)
    for h in (1, 2, 4, 8):
        perm = lane ^ h
        pm = jnp.where((lane & h) == 0, 1.0, -1.0).astype(jnp.float32)
        v = [_dg(x, perm) + pm * x for x in v]
    for hb in (1, 2, 4):
        nv = list(v)
        for j in range(VPR):
            if j & hb == 0:
                nv[j] = v[j] + v[j ^ hb]
            else:
                nv[j] = v[j ^ hb] - v[j]
        v = nv
    return v


def _quant_one(x, bvec, cvec):
    idx = jnp.zeros((L,), jnp.int32)
    for stp in (8, 4, 2, 1):
        bv = _dg(bvec, idx + (stp - 1))
        idx = idx + jnp.where(bv < x, stp, 0).astype(jnp.int32)
    return _dg(cvec, idx)


def _quant_accum(u, q, bvec, cvec):
    num_acc = None
    den_acc = None
    for j in range(VPR):
        r = _quant_one(u[j] * q, bvec, cvec)
        nj = u[j] * r
        dj = r * r
        num_acc = nj if num_acc is None else num_acc + nj
        den_acc = dj if den_acc is None else den_acc + dj
    return _lane_sum(num_acc), _lane_sum(den_acc)


def _sqrt_newton(ss):
    bits = lax.bitcast_convert_type(ss, jnp.int32)
    y = lax.bitcast_convert_type(0x5F3759DF - (bits >> 1), jnp.float32)
    y = y * (1.5 - 0.5 * ss * y * y)
    y = y * (1.5 - 0.5 * ss * y * y)
    return ss * y


def _make_sc_call(rows, ch, interpret=False):
    rpw = rows // NW
    nch = rpw // ch
    mesh = plsc.VectorSubcoreMesh(core_axis_name="c", subcore_axis_name="s",
                                  num_cores=NC, num_subcores=NS)

    @functools.partial(
        pl.kernel,
        out_type=jax.ShapeDtypeStruct((rows * DIM,), jnp.float32),
        mesh=mesh,
        scratch_types=[
            pltpu.VMEM((_CONSTS.size,), jnp.float32),
            pltpu.VMEM((ch * DIM,), jnp.float32),
            pltpu.VMEM((ch * DIM,), jnp.float32),
        ],
        interpret=interpret,
    )
    def sc_fn(x_hbm, c_hbm, o_hbm, cbuf, inb, outb):
        wid = lax.axis_index("s") * NC + lax.axis_index("c")
        base = wid * rpw * DIM

        pltpu.sync_copy(c_hbm, cbuf)
        cvec = cbuf[pl.ds(0, L)]
        bvec = cbuf[pl.ds(L, L)]
        sgn_s = [cbuf[pl.ds(2 * L + L * j, L)] for j in range(VPR)]

        def row_body(i, carry):
            ro = i * DIM
            v = [inb[pl.ds(ro + L * j, L)] for j in range(VPR)]
            ssa = v[0] * v[0]
            for j in range(1, VPR):
                ssa = ssa + v[j] * v[j]
            ss = _lane_sum(ssa)
            u = _fwht_regs([v[j] * sgn_s[j] for j in range(VPR)])
            au = [jnp.abs(t) for t in u]
            mx = au[0]
            sa = au[0]
            for j in range(1, VPR):
                mx = jnp.maximum(mx, au[j])
                sa = sa + au[j]
            maxu = _lane_max(mx)
            sumau = _lane_sum(sa)

            norm = _sqrt_newton(ss) + EPS
            t = EPS * norm
            q1 = 1.0 / (maxu * (1.0 / _MAXC) + t)
            spiky = maxu[0] > 5.0 * (sumau[0] * (1.0 / DIM) + t[0])

            def emit(rec, gain):
                w = _fwht_regs(rec)
                for j in range(VPR):
                    outb[pl.ds(ro + L * j, L)] = w[j] * sgn_s[j] * gain

            def spiky_path():
                rec = [_quant_one(u[j] * q1, bvec, cvec) for j in range(VPR)]
                emit(rec, maxu * (1.0 / _MAXC))

            def normal_path():
                num1, den1 = _quant_accum(u, q1, bvec, cvec)
                g1p = num1 / (den1 + EPS)
                q2 = 1.0 / (g1p + t)
                rec_f = [_quant_one(u[j] * q2, bvec, cvec)
                         for j in range(VPR)]
                num_acc = u[0] * rec_f[0]
                den_acc = rec_f[0] * rec_f[0]
                for j in range(1, VPR):
                    num_acc = num_acc + u[j] * rec_f[j]
                    den_acc = den_acc + rec_f[j] * rec_f[j]
                num2 = _lane_sum(num_acc)
                den2 = _lane_sum(den_acc)
                emit(rec_f, num2 / (den2 + EPS))

            lax.cond(spiky, spiky_path, normal_path)
            return carry

        def chunk_body(ci, carry):
            off = base + ci * (ch * DIM)
            pltpu.sync_copy(x_hbm.at[pl.ds(off, ch * DIM)], inb)
            lax.fori_loop(0, ch, row_body, 0, unroll=1)
            pltpu.sync_copy(outb, o_hbm.at[pl.ds(off, ch * DIM)])
            return carry

        lax.fori_loop(0, nch, chunk_body, 0)

    return sc_fn


def kernel(x):
    shape = x.shape
    rows = x.size // DIM
    xf = x.astype(jnp.float32).reshape(rows * DIM)
    rpw = rows // NW
    ch = 64 if rpw % 64 == 0 else rpw
    out = _make_sc_call(rows, ch)(xf, jnp.asarray(_CONSTS))
    return out.reshape(shape)


# speedup vs baseline: 2.0545x; 1.2378x over previous
import functools
import math

import numpy as np
import jax
import jax.numpy as jnp
from jax import lax
from jax.experimental import pallas as pl
from jax.experimental.pallas import tpu as pltpu
from jax.experimental.pallas import tpu_sc as plsc

DIM = 128
BITS = 4
EPS = 1e-08
L = 16
VPR = DIM // L
NC, NS = 2, 16
NW = NC * NS


def _lm_centroids(bits, iters=100):
    n = 2 ** bits
    xs = np.linspace(-8.0, 8.0, 200001)
    pdf = np.exp(-0.5 * xs ** 2)
    cdf = np.cumsum(pdf)
    cdf = cdf / cdf[-1]
    c = np.interp((np.arange(n) + 0.5) / n, cdf, xs)
    for _ in range(iters):
        b = 0.5 * (c[:-1] + c[1:])
        idx = np.searchsorted(b, xs)
        num = np.bincount(idx, weights=pdf * xs, minlength=n)
        den = np.bincount(idx, weights=pdf, minlength=n)
        c = np.where(den > 1e-12, num / np.maximum(den, 1e-12), c)
    return c.astype(np.float32)


_CEN = _lm_centroids(BITS)
_BND = (0.5 * (_CEN[:-1] + _CEN[1:])).astype(np.float32)
_BND16 = np.concatenate([_BND, [np.float32(np.inf)]])
_MAXC = float(_CEN.max())
_SGN = (np.random.RandomState(42).randint(0, 2, (1, DIM)) * 2 - 1).astype(
    np.float32)[0]
_S = 1.0 / math.sqrt(float(DIM))

_CONSTS = np.concatenate(
    [_CEN, _BND16, (_SGN * np.float32(_S)).astype(np.float32)]
).astype(np.float32)

_GDN = lax.GatherDimensionNumbers(
    offset_dims=(), collapsed_slice_dims=(0,), start_index_map=(0,))


def _dg(vec, idx):
    return lax.gather(vec, idx[:, None], _GDN, (1,),
                      mode=lax.GatherScatterMode.PROMISE_IN_BOUNDS)


def _lane_sum(v):
    lane = lax.iota(jnp.int32, L)
    for h in (1, 2, 4, 8):
        v = v + _dg(v, lane ^ h)
    return v


def _lane_max(v):
    lane = lax.iota(jnp.int32, L)
    for h in (1, 2, 4, 8):
        v = jnp.maximum(v, _dg(v, lane ^ h))
    return v


def _fwht_regs(v):
    lane = lax.iota(jnp.int32, L)
    for h in (1, 2, 4, 8):
        perm = lane ^ h
        pm = jnp.where((lane & h) == 0, 1.0, -1.0).astype(jnp.float32)
        v = [_dg(x, perm) + pm * x for x in v]
    for hb in (1, 2, 4):
        nv = list(v)
        for j in range(VPR):
            if j & hb == 0:
                nv[j] = v[j] + v[j ^ hb]
            else:
                nv[j] = v[j ^ hb] - v[j]
        v = nv
    return v


def _quant_pass(u, q, bvec, cvec):
    recon = []
    num_acc = None
    den_acc = None
    for j in range(VPR):
        xq = u[j] * q
        idx = jnp.zeros((L,), jnp.int32)
        for stp in (8, 4, 2, 1):
            bv = _dg(bvec, idx + (stp - 1))
            idx = idx + jnp.where(bv < xq, stp, 0).astype(jnp.int32)
        r = _dg(cvec, idx)
        recon.append(r)
        nj = u[j] * r
        dj = r * r
        num_acc = nj if num_acc is None else num_acc + nj
        den_acc = dj if den_acc is None else den_acc + dj
    return recon, _lane_sum(num_acc), _lane_sum(den_acc)


def _sqrt_newton(ss):
    bits = lax.bitcast_convert_type(ss, jnp.int32)
    y = lax.bitcast_convert_type(0x5F3759DF - (bits >> 1), jnp.float32)
    for _ in range(3):
        y = y * (1.5 - 0.5 * ss * y * y)
    return ss * y


def _make_sc_call(rows, ch, interpret=False):
    rpw = rows // NW
    nch = rpw // ch
    mesh = plsc.VectorSubcoreMesh(core_axis_name="c", subcore_axis_name="s",
                                  num_cores=NC, num_subcores=NS)

    @functools.partial(
        pl.kernel,
        out_type=jax.ShapeDtypeStruct((rows * DIM,), jnp.float32),
        mesh=mesh,
        scratch_types=[
            pltpu.VMEM((_CONSTS.size,), jnp.float32),
            pltpu.VMEM((ch * DIM,), jnp.float32),
            pltpu.VMEM((ch * DIM,), jnp.float32),
        ],
        interpret=interpret,
    )
    def sc_fn(x_hbm, c_hbm, o_hbm, cbuf, inb, outb):
        wid = lax.axis_index("s") * NC + lax.axis_index("c")
        base = wid * rpw * DIM

        pltpu.sync_copy(c_hbm, cbuf)
        cvec = cbuf[pl.ds(0, L)]
        bvec = cbuf[pl.ds(L, L)]
        sgn_s = [cbuf[pl.ds(2 * L + L * j, L)] for j in range(VPR)]

        def row_body(i, carry):
            ro = i * DIM
            v = [inb[pl.ds(ro + L * j, L)] for j in range(VPR)]
            ssa = v[0] * v[0]
            for j in range(1, VPR):
                ssa = ssa + v[j] * v[j]
            ss = _lane_sum(ssa)
            u = _fwht_regs([v[j] * sgn_s[j] for j in range(VPR)])
            au = [jnp.abs(t) for t in u]
            mx = au[0]
            sa = au[0]
            for j in range(1, VPR):
                mx = jnp.maximum(mx, au[j])
                sa = sa + au[j]
            maxu = _lane_max(mx)
            sumau = _lane_sum(sa)

            norm = _sqrt_newton(ss) + EPS
            t = EPS * norm
            q1 = 1.0 / (maxu * (1.0 / _MAXC) + t)
            rec_u, num1, den1 = _quant_pass(u, q1, bvec, cvec)
            g1p = num1 / (den1 + EPS)
            q2 = 1.0 / (g1p + t)
            rec_f, num2, den2 = _quant_pass(u, q2, bvec, cvec)

            g2 = num2 / (den2 + EPS)
            grms = maxu * (1.0 / _MAXC)
            spiky = maxu > 5.0 * (sumau * (1.0 / DIM) + t)
            gain = jnp.where(spiky, grms, g2)
            rec = [jnp.where(spiky, rec_u[j], rec_f[j]) for j in range(VPR)]

            w = _fwht_regs(rec)
            for j in range(VPR):
                outb[pl.ds(ro + L * j, L)] = w[j] * sgn_s[j] * gain
            return carry

        def chunk_body(ci, carry):
            off = base + ci * (ch * DIM)
            pltpu.sync_copy(x_hbm.at[pl.ds(off, ch * DIM)], inb)
            lax.fori_loop(0, ch, row_body, 0)
            pltpu.sync_copy(outb, o_hbm.at[pl.ds(off, ch * DIM)])
            return carry

        lax.fori_loop(0, nch, chunk_body, 0)

    return sc_fn


def kernel(x):
    shape = x.shape
    rows = x.size // DIM
    xf = x.astype(jnp.float32).reshape(rows * DIM)
    rpw = rows // NW
    ch = 64 if rpw % 64 == 0 else rpw
    out = _make_sc_call(rows, ch)(xf, jnp.asarray(_CONSTS))
    return out.reshape(shape)


# speedup vs baseline: 2.1880x; 1.0650x over previous
import functools
import math

import numpy as np
import jax
import jax.numpy as jnp
from jax import lax
from jax.experimental import pallas as pl
from jax.experimental.pallas import tpu as pltpu
from jax.experimental.pallas import tpu_sc as plsc

DIM = 128
BITS = 4
EPS = 1e-08
L = 16
VPR = DIM // L
NC, NS = 2, 16
NW = NC * NS


def _lm_centroids(bits, iters=100):
    n = 2 ** bits
    xs = np.linspace(-8.0, 8.0, 200001)
    pdf = np.exp(-0.5 * xs ** 2)
    cdf = np.cumsum(pdf)
    cdf = cdf / cdf[-1]
    c = np.interp((np.arange(n) + 0.5) / n, cdf, xs)
    for _ in range(iters):
        b = 0.5 * (c[:-1] + c[1:])
        idx = np.searchsorted(b, xs)
        num = np.bincount(idx, weights=pdf * xs, minlength=n)
        den = np.bincount(idx, weights=pdf, minlength=n)
        c = np.where(den > 1e-12, num / np.maximum(den, 1e-12), c)
    return c.astype(np.float32)


_CEN = _lm_centroids(BITS)
_BND = (0.5 * (_CEN[:-1] + _CEN[1:])).astype(np.float32)
_BND16 = np.concatenate([_BND, [np.float32(np.inf)]])
_MAXC = float(_CEN.max())
_SGN = (np.random.RandomState(42).randint(0, 2, (1, DIM)) * 2 - 1).astype(
    np.float32)[0]
_S = 1.0 / math.sqrt(float(DIM))

_CONSTS = np.concatenate(
    [_CEN, _BND16, (_SGN * np.float32(_S)).astype(np.float32)]
).astype(np.float32)

_GDN = lax.GatherDimensionNumbers(
    offset_dims=(), collapsed_slice_dims=(0,), start_index_map=(0,))


def _dg(vec, idx):
    return lax.gather(vec, idx[:, None], _GDN, (1,),
                      mode=lax.GatherScatterMode.PROMISE_IN_BOUNDS)


def _lane_sum(v):
    lane = lax.iota(jnp.int32, L)
    for h in (1, 2, 4, 8):
        v = v + _dg(v, lane ^ h)
    return v


def _lane_max(v):
    lane = lax.iota(jnp.int32, L)
    for h in (1, 2, 4, 8):
        v = jnp.maximum(v, _dg(v, lane ^ h))
    return v


def _fwht_regs(v):
    lane = lax.iota(jnp.int32, L)
    for h in (1, 2, 4, 8):
        perm = lane ^ h
        pm = jnp.where((lane & h) == 0, 1.0, -1.0).astype(jnp.float32)
        v = [_dg(x, perm) + pm * x for x in v]
    for hb in (1, 2, 4):
        nv = list(v)
        for j in range(VPR):
            if j & hb == 0:
                nv[j] = v[j] + v[j ^ hb]
            else:
                nv[j] = v[j ^ hb] - v[j]
        v = nv
    return v


def _quant_pass(u, q, bvec, cvec):
    recon = []
    num_acc = None
    den_acc = None
    for j in range(VPR):
        xq = u[j] * q
        idx = jnp.zeros((L,), jnp.int32)
        for stp in (8, 4, 2, 1):
            bv = _dg(bvec, idx + (stp - 1))
            idx = idx + jnp.where(bv < xq, stp, 0).astype(jnp.int32)
        r = _dg(cvec, idx)
        recon.append(r)
        nj = u[j] * r
        dj = r * r
        num_acc = nj if num_acc is None else num_acc + nj
        den_acc = dj if den_acc is None else den_acc + dj
    return recon, _lane_sum(num_acc), _lane_sum(den_acc)


def _make_sc_call(rows, ch, interpret=False):
    rpw = rows // NW
    nch = rpw // ch
    mesh = plsc.VectorSubcoreMesh(core_axis_name="c", subcore_axis_name="s",
                                  num_cores=NC, num_subcores=NS)

    @functools.partial(
        pl.kernel,
        out_type=jax.ShapeDtypeStruct((rows * DIM,), jnp.float32),
        mesh=mesh,
        scratch_types=[
            pltpu.VMEM((_CONSTS.size,), jnp.float32),
            pltpu.VMEM((ch * DIM,), jnp.float32),
            pltpu.VMEM((ch * DIM,), jnp.float32),
        ],
        interpret=interpret,
    )
    def sc_fn(x_hbm, c_hbm, o_hbm, cbuf, inb, outb):
        wid = lax.axis_index("s") * NC + lax.axis_index("c")
        base = wid * rpw * DIM

        pltpu.sync_copy(c_hbm, cbuf)
        cvec = cbuf[pl.ds(0, L)]
        bvec = cbuf[pl.ds(L, L)]
        sgn_s = [cbuf[pl.ds(2 * L + L * j, L)] for j in range(VPR)]

        def row_body(i, carry):
            ro = i * DIM
            v = [inb[pl.ds(ro + L * j, L)] for j in range(VPR)]
            u = _fwht_regs([v[j] * sgn_s[j] for j in range(VPR)])
            au = [jnp.abs(t) for t in u]
            mx = au[0]
            sa = au[0]
            for j in range(1, VPR):
                mx = jnp.maximum(mx, au[j])
                sa = sa + au[j]
            maxu = _lane_max(mx)
            sumau = _lane_sum(sa)

            t = EPS * (maxu * math.sqrt(float(DIM)) + EPS)
            q1 = 1.0 / (maxu * (1.0 / _MAXC) + t)
            rec_u, num1, den1 = _quant_pass(u, q1, bvec, cvec)
            g1p = num1 / (den1 + EPS)
            q2 = 1.0 / (g1p + t)
            rec_f, num2, den2 = _quant_pass(u, q2, bvec, cvec)

            g2 = num2 / (den2 + EPS)
            grms = maxu * (1.0 / _MAXC)
            spiky = maxu > 5.0 * (sumau * (1.0 / DIM) + t)
            gain = jnp.where(spiky, grms, g2)
            rec = [jnp.where(spiky, rec_u[j], rec_f[j]) for j in range(VPR)]

            w = _fwht_regs(rec)
            for j in range(VPR):
                outb[pl.ds(ro + L * j, L)] = w[j] * sgn_s[j] * gain
            return carry

        def chunk_body(ci, carry):
            off = base + ci * (ch * DIM)
            pltpu.sync_copy(x_hbm.at[pl.ds(off, ch * DIM)], inb)
            lax.fori_loop(0, ch, row_body, 0)
            pltpu.sync_copy(outb, o_hbm.at[pl.ds(off, ch * DIM)])
            return carry

        lax.fori_loop(0, nch, chunk_body, 0)

    return sc_fn


def kernel(x):
    shape = x.shape
    rows = x.size // DIM
    xf = x.astype(jnp.float32).reshape(rows * DIM)
    rpw = rows // NW
    ch = 64 if rpw % 64 == 0 else rpw
    out = _make_sc_call(rows, ch)(xf, jnp.asarray(_CONSTS))
    return out.reshape(shape)
